# baseline (speedup 1.0000x reference)
import functools

import jax
import jax.numpy as jnp
from jax import lax
from jax.experimental import pallas as pl
from jax.experimental.pallas import tpu as pltpu
from jax.experimental.pallas import tpu_sc as plsc

F = 26
C = 64
L = 16
NCH = C // L
P = F * (F - 1) // 2
NC = 2
NS = 16
NW = NC * NS

_BLOCKS = [(0, 6), (6, 6), (12, 6), (18, 6), (24, 2)]


def _seg_start(i):
    return i * (F - 1) - i * (i - 1) // 2


def _emit_sample(xv, ov):
    for i0, r in _BLOCKS:
        rows = [[xv[pl.ds((i0 + a) * C + c * L, L)] for c in range(NCH)]
                for a in range(r)]
        for a in range(r):
            i = i0 + a
            for b in range(a + 1, r):
                j = i0 + b
                pos = _seg_start(i) + (j - i - 1)
                for c in range(NCH):
                    ov[pl.ds(pos * C + c * L, L)] = rows[a][c] * rows[b][c]
        j0 = i0 + r
        if j0 >= F:
            continue

        def body(j, carry, i0=i0, r=r, rows=rows):
            jc = j * C
            xj = [xv[pl.ds(jc + c * L, L)] for c in range(NCH)]
            for a in range(r):
                i = i0 + a
                stat = (_seg_start(i) - i - 1) * C
                for c in range(NCH):
                    ov[pl.ds(stat + jc + c * L, L)] = rows[a][c] * xj[c]
            return carry

        lax.fori_loop(j0, F, body, 0)


def _sc_pairs(xf):
    B = xf.shape[0]
    assert B % NW == 0, B
    n = B // NW
    assert n % 2 == 0, n

    mesh = plsc.VectorSubcoreMesh(core_axis_name="c", subcore_axis_name="s")

    @functools.partial(
        pl.kernel,
        mesh=mesh,
        out_type=jax.ShapeDtypeStruct((B, P * C), jnp.float32),
        scratch_types=[
            pltpu.VMEM((2, F * C), jnp.float32),
            pltpu.VMEM((2, P * C), jnp.float32),
            pltpu.SemaphoreType.DMA,
            pltpu.SemaphoreType.DMA,
            pltpu.SemaphoreType.DMA,
            pltpu.SemaphoreType.DMA,
        ],
    )
    def k(x_hbm, out_hbm, xin, oub, lsem0, lsem1, ssem0, ssem1):
        lsem = (lsem0, lsem1)
        ssem = (ssem0, ssem1)
        wid = lax.axis_index("s") * NC + lax.axis_index("c")
        base = wid * n

        for par in range(2):
            pltpu.make_async_copy(
                x_hbm.at[base + par], xin.at[par], lsem[par]).start()

        def step(t, carry):
            for par in range(2):
                s = 2 * t + par
                pltpu.make_async_copy(
                    x_hbm.at[base + s], xin.at[par], lsem[par]).wait()
                @pl.when(s >= 2)
                def _():
                    pltpu.make_async_copy(
                        oub.at[par], out_hbm.at[base + s], ssem[par]).wait()

                _emit_sample(xin.at[par], oub.at[par])

                pltpu.make_async_copy(
                    oub.at[par], out_hbm.at[base + s], ssem[par]).start()
                nxt = jnp.minimum(s + 2, n - 1)
                pltpu.make_async_copy(
                    x_hbm.at[base + nxt], xin.at[par], lsem[par]).start()
            return carry

        lax.fori_loop(0, n // 2, step, 0)

        for par in range(2):
            pltpu.make_async_copy(
                oub.at[par], out_hbm.at[base + n - 2 + par],
                ssem[par]).wait()
            pltpu.make_async_copy(
                x_hbm.at[base + n - 1], xin.at[par], lsem[par]).wait()

    return k(xf)


def _tc_pairs(x):
    B = x.shape[0]
    BB = 64
    assert B % BB == 0

    def body(x_ref, o_ref):
        xb = x_ref[...]
        off = 0
        for i in range(F - 1):
            n = F - 1 - i
            o_ref[:, off:off + n, :] = xb[:, i:i + 1, :] * xb[:, i + 1:, :]
            off += n

    return pl.pallas_call(
        body,
        grid=(B // BB,),
        in_specs=[pl.BlockSpec((BB, F, C), lambda b: (b, 0, 0))],
        out_specs=pl.BlockSpec((BB, P, C), lambda b: (b, 0, 0)),
        out_shape=jax.ShapeDtypeStruct((B, P, C), jnp.float32),
    )(x)


def kernel(x):
    B, f, c = x.shape
    assert (f, c) == (F, C), (f, c)
    return _tc_pairs(x)


# speedup vs baseline: 1.0840x; 1.0840x over previous
import functools

import jax
import jax.numpy as jnp
from jax import lax
from jax.experimental import pallas as pl
from jax.experimental.pallas import tpu as pltpu
from jax.experimental.pallas import tpu_sc as plsc

F = 26
C = 64
L = 16
NCH = C // L
P = F * (F - 1) // 2
NC = 2
NS = 16
NW = NC * NS

_BLOCKS = [(0, 6), (6, 6), (12, 6), (18, 6), (24, 2)]


def _seg_start(i):
    return i * (F - 1) - i * (i - 1) // 2


def _emit_sample(xv, ov):
    for i0, r in _BLOCKS:
        rows = [[xv[pl.ds((i0 + a) * C + c * L, L)] for c in range(NCH)]
                for a in range(r)]
        for a in range(r):
            i = i0 + a
            for b in range(a + 1, r):
                j = i0 + b
                pos = _seg_start(i) + (j - i - 1)
                for c in range(NCH):
                    ov[pl.ds(pos * C + c * L, L)] = rows[a][c] * rows[b][c]
        j0 = i0 + r
        if j0 >= F:
            continue

        def body(j, carry, i0=i0, r=r, rows=rows):
            jc = j * C
            xj = [xv[pl.ds(jc + c * L, L)] for c in range(NCH)]
            for a in range(r):
                i = i0 + a
                stat = (_seg_start(i) - i - 1) * C
                for c in range(NCH):
                    ov[pl.ds(stat + jc + c * L, L)] = rows[a][c] * xj[c]
            return carry

        lax.fori_loop(j0, F, body, 0)


def _sc_pairs(xf):
    B = xf.shape[0]
    assert B % NW == 0, B
    n = B // NW
    assert n % 2 == 0, n

    mesh = plsc.VectorSubcoreMesh(core_axis_name="c", subcore_axis_name="s")

    @functools.partial(
        pl.kernel,
        mesh=mesh,
        out_type=jax.ShapeDtypeStruct((B, P * C), jnp.float32),
        scratch_types=[
            pltpu.VMEM((2, F * C), jnp.float32),
            pltpu.VMEM((2, P * C), jnp.float32),
            pltpu.SemaphoreType.DMA,
            pltpu.SemaphoreType.DMA,
            pltpu.SemaphoreType.DMA,
            pltpu.SemaphoreType.DMA,
        ],
    )
    def k(x_hbm, out_hbm, xin, oub, lsem0, lsem1, ssem0, ssem1):
        lsem = (lsem0, lsem1)
        ssem = (ssem0, ssem1)
        wid = lax.axis_index("s") * NC + lax.axis_index("c")
        base = wid * n

        for par in range(2):
            pltpu.make_async_copy(
                x_hbm.at[base + par], xin.at[par], lsem[par]).start()

        def step(t, carry):
            for par in range(2):
                s = 2 * t + par
                pltpu.make_async_copy(
                    x_hbm.at[base + s], xin.at[par], lsem[par]).wait()
                @pl.when(s >= 2)
                def _():
                    pltpu.make_async_copy(
                        oub.at[par], out_hbm.at[base + s], ssem[par]).wait()

                _emit_sample(xin.at[par], oub.at[par])

                pltpu.make_async_copy(
                    oub.at[par], out_hbm.at[base + s], ssem[par]).start()
                nxt = jnp.minimum(s + 2, n - 1)
                pltpu.make_async_copy(
                    x_hbm.at[base + nxt], xin.at[par], lsem[par]).start()
            return carry

        lax.fori_loop(0, n // 2, step, 0)

        for par in range(2):
            pltpu.make_async_copy(
                oub.at[par], out_hbm.at[base + n - 2 + par],
                ssem[par]).wait()
            pltpu.make_async_copy(
                x_hbm.at[base + n - 1], xin.at[par], lsem[par]).wait()

    return k(xf)


def _tc_pairs(x):
    B = x.shape[0]
    BB = 64
    assert B % BB == 0

    def body(x_ref, o_ref):
        xb = x_ref[...]
        off = 0
        for i in range(F - 1):
            n = F - 1 - i
            o_ref[:, off:off + n, :] = xb[:, i:i + 1, :] * xb[:, i + 1:, :]
            off += n

    return pl.pallas_call(
        body,
        grid=(B // BB,),
        in_specs=[pl.BlockSpec((BB, F, C), lambda b: (b, 0, 0))],
        out_specs=pl.BlockSpec((BB, P, C), lambda b: (b, 0, 0)),
        out_shape=jax.ShapeDtypeStruct((B, P, C), jnp.float32),
    )(x)


def kernel(x):
    B, f, c = x.shape
    assert (f, c) == (F, C), (f, c)
    K = 8
    assert B % K == 0
    bc = B // K
    xf = x.reshape(B, F * C)
    parts = [
        _sc_pairs(xf[k * bc:(k + 1) * bc]).reshape(bc, P, C)
        for k in range(K)
    ]
    return jnp.concatenate(parts, axis=0)


# speedup vs baseline: 1.2320x; 1.1365x over previous
import functools

import jax
import jax.numpy as jnp
from jax import lax
from jax.experimental import pallas as pl
from jax.experimental.pallas import tpu as pltpu
from jax.experimental.pallas import tpu_sc as plsc

F = 26
C = 64
L = 16
NCH = C // L
P = F * (F - 1) // 2
NC = 2
NS = 16
NW = NC * NS

_BLOCKS = [(0, 6), (6, 6), (12, 6), (18, 6), (24, 2)]


def _seg_start(i):
    return i * (F - 1) - i * (i - 1) // 2


def _emit_sample(xv, ov):
    for i0, r in _BLOCKS:
        rows = [[xv[pl.ds((i0 + a) * C + c * L, L)] for c in range(NCH)]
                for a in range(r)]
        for a in range(r):
            i = i0 + a
            for b in range(a + 1, r):
                j = i0 + b
                pos = _seg_start(i) + (j - i - 1)
                for c in range(NCH):
                    ov[pl.ds(pos * C + c * L, L)] = rows[a][c] * rows[b][c]
        j0 = i0 + r
        if j0 >= F:
            continue

        def body(j, carry, i0=i0, r=r, rows=rows):
            jc = j * C
            xj = [xv[pl.ds(jc + c * L, L)] for c in range(NCH)]
            for a in range(r):
                i = i0 + a
                stat = (_seg_start(i) - i - 1) * C
                for c in range(NCH):
                    ov[pl.ds(stat + jc + c * L, L)] = rows[a][c] * xj[c]
            return carry

        lax.fori_loop(j0, F, body, 0)


def _sc_pairs(xf):
    B = xf.shape[0]
    assert B % NW == 0, B
    n = B // NW
    assert n % 2 == 0, n

    mesh = plsc.VectorSubcoreMesh(core_axis_name="c", subcore_axis_name="s")

    @functools.partial(
        pl.kernel,
        mesh=mesh,
        out_type=jax.ShapeDtypeStruct((B, P * C), jnp.float32),
        scratch_types=[
            pltpu.VMEM((2, F * C), jnp.float32),
            pltpu.VMEM((2, P * C), jnp.float32),
            pltpu.SemaphoreType.DMA,
            pltpu.SemaphoreType.DMA,
            pltpu.SemaphoreType.DMA,
            pltpu.SemaphoreType.DMA,
        ],
    )
    def k(x_hbm, out_hbm, xin, oub, lsem0, lsem1, ssem0, ssem1):
        lsem = (lsem0, lsem1)
        ssem = (ssem0, ssem1)
        wid = lax.axis_index("s") * NC + lax.axis_index("c")
        base = wid * n

        for par in range(2):
            pltpu.make_async_copy(
                x_hbm.at[base + par], xin.at[par], lsem[par]).start()

        def step(t, carry):
            for par in range(2):
                s = 2 * t + par
                pltpu.make_async_copy(
                    x_hbm.at[base + s], xin.at[par], lsem[par]).wait()
                @pl.when(s >= 2)
                def _():
                    pltpu.make_async_copy(
                        oub.at[par], out_hbm.at[base + s], ssem[par]).wait()

                _emit_sample(xin.at[par], oub.at[par])

                pltpu.make_async_copy(
                    oub.at[par], out_hbm.at[base + s], ssem[par]).start()
                nxt = jnp.minimum(s + 2, n - 1)
                pltpu.make_async_copy(
                    x_hbm.at[base + nxt], xin.at[par], lsem[par]).start()
            return carry

        lax.fori_loop(0, n // 2, step, 0)

        for par in range(2):
            pltpu.make_async_copy(
                oub.at[par], out_hbm.at[base + n - 2 + par],
                ssem[par]).wait()
            pltpu.make_async_copy(
                x_hbm.at[base + n - 1], xin.at[par], lsem[par]).wait()

    return k(xf)


def kernel(x):
    B, f, c = x.shape
    assert (f, c) == (F, C), (f, c)
    out = _sc_pairs(x.reshape(B, F * C))
    return out.reshape(B, P, C)


# speedup vs baseline: 1.4076x; 1.1426x over previous
import functools

import jax
import jax.numpy as jnp
from jax import lax
from jax.experimental import pallas as pl
from jax.experimental.pallas import tpu as pltpu
from jax.experimental.pallas import tpu_sc as plsc

F = 26
C = 64
L = 16
NCH = C // L
P = F * (F - 1) // 2
NC = 2
NS = 16
NW = NC * NS

_BLOCKS = [(0, 6), (6, 6), (12, 6), (18, 6), (24, 2)]


def _seg_start(i):
    return i * (F - 1) - i * (i - 1) // 2


def _emit_sample(xv, ov):
    for i0, r in _BLOCKS:
        rows = [[xv[pl.ds((i0 + a) * C + c * L, L)] for c in range(NCH)]
                for a in range(r)]
        for a in range(r):
            i = i0 + a
            for b in range(a + 1, r):
                j = i0 + b
                pos = _seg_start(i) + (j - i - 1)
                for c in range(NCH):
                    ov[pl.ds(pos * C + c * L, L)] = rows[a][c] * rows[b][c]
        j0 = i0 + r
        if j0 >= F:
            continue

        def body(j, carry, i0=i0, r=r, rows=rows):
            jc = j * C
            xj = [xv[pl.ds(jc + c * L, L)] for c in range(NCH)]
            for a in range(r):
                i = i0 + a
                stat = (_seg_start(i) - i - 1) * C
                for c in range(NCH):
                    ov[pl.ds(stat + jc + c * L, L)] = rows[a][c] * xj[c]
            return carry

        lax.fori_loop(j0, F, body, 0)


def _sc_pairs(xf):
    B = xf.shape[0]
    assert B % NW == 0, B
    n = B // NW
    assert n % 2 == 0, n

    mesh = plsc.VectorSubcoreMesh(core_axis_name="c", subcore_axis_name="s")

    @functools.partial(
        pl.kernel,
        mesh=mesh,
        out_type=jax.ShapeDtypeStruct((B, P * C), jnp.float32),
        scratch_types=[
            pltpu.VMEM((2, F * C), jnp.float32),
            pltpu.VMEM((2, P * C), jnp.float32),
            pltpu.SemaphoreType.DMA,
            pltpu.SemaphoreType.DMA,
            pltpu.SemaphoreType.DMA,
            pltpu.SemaphoreType.DMA,
        ],
    )
    def k(x_hbm, out_hbm, xin, oub, lsem0, lsem1, ssem0, ssem1):
        lsem = (lsem0, lsem1)
        ssem = (ssem0, ssem1)
        wid = lax.axis_index("s") * NC + lax.axis_index("c")
        base = wid * n

        for par in range(2):
            pltpu.make_async_copy(
                x_hbm.at[base + par], xin.at[par], lsem[par]).start()

        def step(t, carry):
            for par in range(2):
                s = 2 * t + par
                pltpu.make_async_copy(
                    x_hbm.at[base + s], xin.at[par], lsem[par]).wait()
                @pl.when(s >= 2)
                def _():
                    pltpu.make_async_copy(
                        oub.at[par], out_hbm.at[base + s], ssem[par]).wait()

                _emit_sample(xin.at[par], oub.at[par])

                pltpu.make_async_copy(
                    oub.at[par], out_hbm.at[base + s], ssem[par]).start()
                nxt = jnp.minimum(s + 2, n - 1)
                pltpu.make_async_copy(
                    x_hbm.at[base + nxt], xin.at[par], lsem[par]).start()
            return carry

        lax.fori_loop(0, n // 2, step, 0)

        for par in range(2):
            pltpu.make_async_copy(
                oub.at[par], out_hbm.at[base + n - 2 + par],
                ssem[par]).wait()
            pltpu.make_async_copy(
                x_hbm.at[base + n - 1], xin.at[par], lsem[par]).wait()

    return k(xf)


def _sc_pairs2(xf):
    B = xf.shape[0]
    assert B % NW == 0, B
    n = B // NW
    assert n % 4 == 0, n
    g = n // 2

    mesh = plsc.VectorSubcoreMesh(core_axis_name="c", subcore_axis_name="s")

    @functools.partial(
        pl.kernel,
        mesh=mesh,
        out_type=jax.ShapeDtypeStruct((B, P * C), jnp.float32),
        scratch_types=[
            pltpu.VMEM((2, 2, F * C), jnp.float32),
            pltpu.VMEM((2, 2, P * C), jnp.float32),
            pltpu.SemaphoreType.DMA,
            pltpu.SemaphoreType.DMA,
            pltpu.SemaphoreType.DMA,
            pltpu.SemaphoreType.DMA,
        ],
    )
    def k(x_hbm, out_hbm, xin, oub, lsem0, lsem1, ssem0, ssem1):
        lsem = (lsem0, lsem1)
        ssem = (ssem0, ssem1)
        wid = lax.axis_index("s") * NC + lax.axis_index("c")
        base = wid * n

        for par in range(2):
            pltpu.make_async_copy(
                x_hbm.at[pl.ds(base + 2 * par, 2)], xin.at[par],
                lsem[par]).start()

        def step(t, carry):
            for par in range(2):
                q = 2 * t + par
                row = base + 2 * q
                pltpu.make_async_copy(
                    x_hbm.at[pl.ds(row, 2)], xin.at[par], lsem[par]).wait()

                @pl.when(q >= 2)
                def _():
                    pltpu.make_async_copy(
                        oub.at[par], out_hbm.at[pl.ds(row, 2)],
                        ssem[par]).wait()

                for u in range(2):
                    _emit_sample(xin.at[par, u], oub.at[par, u])

                pltpu.make_async_copy(
                    oub.at[par], out_hbm.at[pl.ds(row, 2)],
                    ssem[par]).start()
                nxt = jnp.minimum(2 * (q + 2), n - 2)
                pltpu.make_async_copy(
                    x_hbm.at[pl.ds(base + nxt, 2)], xin.at[par],
                    lsem[par]).start()
            return carry

        lax.fori_loop(0, g // 2, step, 0)

        for par in range(2):
            pltpu.make_async_copy(
                oub.at[par],
                out_hbm.at[pl.ds(base + n - 4 + 2 * par, 2)],
                ssem[par]).wait()
            pltpu.make_async_copy(
                x_hbm.at[pl.ds(base + n - 2, 2)], xin.at[par],
                lsem[par]).wait()

    return k(xf)


def kernel(x):
    B, f, c = x.shape
    assert (f, c) == (F, C), (f, c)
    out = _sc_pairs2(x.reshape(B, F * C))
    return out.reshape(B, P, C)
